# baseline (speedup 1.0000x reference)
import functools

import jax
import jax.numpy as jnp
from jax import lax
from jax.experimental import pallas as pl
from jax.experimental.pallas import tpu as pltpu
from jax.experimental.pallas import tpu_sc as plsc

B = 16384
F = 64
_NC = 2
_NS = 16
_NW = _NC * _NS
_BPW = B // _NW
_CH = 128
_NCH = _BPW // _CH


def _sc_gather(user, item, tug, tig, tum, tim):
    mesh = plsc.VectorSubcoreMesh(core_axis_name="c", subcore_axis_name="s")

    @functools.partial(
        pl.kernel,
        mesh=mesh,
        out_type=[jax.ShapeDtypeStruct((B, F), jnp.float32)] * 4,
        scratch_types=[
            pltpu.VMEM((_CH,), jnp.int32),
            pltpu.VMEM((_CH,), jnp.int32),
            pltpu.VMEM((_CH, F), jnp.float32),
            pltpu.VMEM((_CH, F), jnp.float32),
            pltpu.VMEM((_CH, F), jnp.float32),
            pltpu.VMEM((_CH, F), jnp.float32),
            pltpu.SemaphoreType.DMA,
        ],
        compiler_params=pltpu.CompilerParams(use_tc_tiling_on_sc=False),
    )
    def k(user_h, item_h, tug_h, tig_h, tum_h, tim_h,
          oug_h, oig_h, oum_h, oim_h,
          idxu, idxi, bug, big, bum, bim, sem):
        wid = lax.axis_index("s") * _NC + lax.axis_index("c")
        base = wid * _BPW

        def chunk(c, carry):
            off = base + c * _CH
            pltpu.sync_copy(user_h.at[pl.ds(off, _CH)], idxu)
            pltpu.sync_copy(item_h.at[pl.ds(off, _CH)], idxi)
            cug = pltpu.async_copy(tug_h.at[idxu], bug, sem)
            cig = pltpu.async_copy(tig_h.at[idxi], big, sem)
            cum = pltpu.async_copy(tum_h.at[idxu], bum, sem)
            cim = pltpu.async_copy(tim_h.at[idxi], bim, sem)
            cug.wait()
            cig.wait()
            cum.wait()
            cim.wait()
            pltpu.sync_copy(bug, oug_h.at[pl.ds(off, _CH)])
            pltpu.sync_copy(big, oig_h.at[pl.ds(off, _CH)])
            pltpu.sync_copy(bum, oum_h.at[pl.ds(off, _CH)])
            pltpu.sync_copy(bim, oim_h.at[pl.ds(off, _CH)])
            return carry

        lax.fori_loop(0, _NCH, chunk, 0)

    return k(user, item, tug, tig, tum, tim)


def _tc_head(eug, eig, eum, eim, w1a_t, w1b_t, b1r, wg, wh, bp11):
    BB = 2048

    def body(eug_r, eig_r, eum_r, eim_r, w1a_r, w1b_r, b1_r, wg_r, wh_r,
             bp_r, out_r):
        h = jnp.dot(eum_r[...], w1a_r[...], preferred_element_type=jnp.float32)
        h = h + jnp.dot(eim_r[...], w1b_r[...],
                        preferred_element_type=jnp.float32)
        h = jnp.maximum(h + b1_r[...], 0.0)
        gmf = eug_r[...] * eig_r[...]
        logit = (jnp.sum(gmf * wg_r[...], axis=1, keepdims=True)
                 + jnp.sum(h * wh_r[...], axis=1, keepdims=True)
                 + bp_r[...])
        out_r[...] = 1.0 / (1.0 + jnp.exp(-logit))

    batch_spec = pl.BlockSpec((BB, F), lambda i: (i, 0))
    full_spec = pl.BlockSpec((F, F), lambda i: (0, 0))
    row_spec = pl.BlockSpec((1, F), lambda i: (0, 0))
    return pl.pallas_call(
        body,
        grid=(B // BB,),
        in_specs=[batch_spec, batch_spec, batch_spec, batch_spec,
                  full_spec, full_spec, row_spec, row_spec, row_spec,
                  pl.BlockSpec((1, 1), lambda i: (0, 0))],
        out_specs=pl.BlockSpec((BB, 1), lambda i: (i, 0)),
        out_shape=jax.ShapeDtypeStruct((B, 1), jnp.float32),
    )(eug, eig, eum, eim, w1a_t, w1b_t, b1r, wg, wh, bp11)


def kernel(user, item, embed_user_GMF, embed_item_GMF, embed_user_MLP,
           embed_item_MLP, W1, b1, Wp, bp):
    user = user.astype(jnp.int32)
    item = item.astype(jnp.int32)
    eug, eig, eum, eim = _sc_gather(user, item, embed_user_GMF,
                                    embed_item_GMF, embed_user_MLP,
                                    embed_item_MLP)
    w1a_t = W1[:, :F].T
    w1b_t = W1[:, F:].T
    b1r = b1.reshape(1, F)
    wg = Wp[0, :F].reshape(1, F)
    wh = Wp[0, F:].reshape(1, F)
    bp11 = bp.reshape(1, 1)
    out = _tc_head(eug, eig, eum, eim, w1a_t, w1b_t, b1r, wg, wh, bp11)
    return out.reshape(B)


# speedup vs baseline: 1.0004x; 1.0004x over previous
import functools

import jax
import jax.numpy as jnp
from jax import lax
from jax.experimental import pallas as pl
from jax.experimental.pallas import tpu as pltpu
from jax.experimental.pallas import tpu_sc as plsc

B = 16384
F = 64
_NC = 2
_NS = 16
_NW = _NC * _NS
_BPW = B // _NW
_CH = 128
_NCH = _BPW // _CH


def _sc_gather(pu, pi, tug2, tig2, tum2, tim2):
    mesh = plsc.VectorSubcoreMesh(core_axis_name="c", subcore_axis_name="s")

    @functools.partial(
        pl.kernel,
        mesh=mesh,
        out_type=[jax.ShapeDtypeStruct((B, 2 * F), jnp.float32)] * 4,
        scratch_types=[
            pltpu.VMEM((_CH,), jnp.int32),
            pltpu.VMEM((_CH,), jnp.int32),
            pltpu.VMEM((_CH, 2 * F), jnp.float32),
            pltpu.VMEM((_CH, 2 * F), jnp.float32),
            pltpu.VMEM((_CH, 2 * F), jnp.float32),
            pltpu.VMEM((_CH, 2 * F), jnp.float32),
            pltpu.SemaphoreType.DMA,
        ],
    )
    def k(pu_h, pi_h, tug_h, tig_h, tum_h, tim_h,
          oug_h, oig_h, oum_h, oim_h,
          idxu, idxi, bug, big, bum, bim, sem):
        wid = lax.axis_index("s") * _NC + lax.axis_index("c")
        base = wid * _BPW

        def chunk(c, carry):
            off = base + c * _CH
            pltpu.sync_copy(pu_h.at[pl.ds(off, _CH)], idxu)
            pltpu.sync_copy(pi_h.at[pl.ds(off, _CH)], idxi)
            cug = pltpu.async_copy(tug_h.at[idxu], bug, sem)
            cig = pltpu.async_copy(tig_h.at[idxi], big, sem)
            cum = pltpu.async_copy(tum_h.at[idxu], bum, sem)
            cim = pltpu.async_copy(tim_h.at[idxi], bim, sem)
            cug.wait()
            cig.wait()
            cum.wait()
            cim.wait()
            pltpu.sync_copy(bug, oug_h.at[pl.ds(off, _CH)])
            pltpu.sync_copy(big, oig_h.at[pl.ds(off, _CH)])
            pltpu.sync_copy(bum, oum_h.at[pl.ds(off, _CH)])
            pltpu.sync_copy(bim, oim_h.at[pl.ds(off, _CH)])
            return carry

        lax.fori_loop(0, _NCH, chunk, 0)

    return k(pu, pi, tug2, tig2, tum2, tim2)


def _tc_head(eug2, eig2, eum2, eim2, par_u, par_i, w1a_t, w1b_t, b1r, wg, wh,
             bp11):
    BB = 2048

    def body(eug_r, eig_r, eum_r, eim_r, pu_r, pi_r, w1a_r, w1b_r, b1_r,
             wg_r, wh_r, bp_r, out_r):
        sel_u = pu_r[...] == 0
        sel_i = pi_r[...] == 0
        eug = jnp.where(sel_u, eug_r[:, :F], eug_r[:, F:])
        eum = jnp.where(sel_u, eum_r[:, :F], eum_r[:, F:])
        eig = jnp.where(sel_i, eig_r[:, :F], eig_r[:, F:])
        eim = jnp.where(sel_i, eim_r[:, :F], eim_r[:, F:])
        h = jnp.dot(eum, w1a_r[...], preferred_element_type=jnp.float32)
        h = h + jnp.dot(eim, w1b_r[...], preferred_element_type=jnp.float32)
        h = jnp.maximum(h + b1_r[...], 0.0)
        gmf = eug * eig
        logit = (jnp.sum(gmf * wg_r[...], axis=1, keepdims=True)
                 + jnp.sum(h * wh_r[...], axis=1, keepdims=True)
                 + bp_r[...])
        out_r[...] = 1.0 / (1.0 + jnp.exp(-logit))

    batch_spec = pl.BlockSpec((BB, 2 * F), lambda i: (i, 0))
    par_spec = pl.BlockSpec((BB, 1), lambda i: (i, 0))
    full_spec = pl.BlockSpec((F, F), lambda i: (0, 0))
    row_spec = pl.BlockSpec((1, F), lambda i: (0, 0))
    return pl.pallas_call(
        body,
        grid=(B // BB,),
        in_specs=[batch_spec, batch_spec, batch_spec, batch_spec,
                  par_spec, par_spec,
                  full_spec, full_spec, row_spec, row_spec, row_spec,
                  pl.BlockSpec((1, 1), lambda i: (0, 0))],
        out_specs=pl.BlockSpec((BB, 1), lambda i: (i, 0)),
        out_shape=jax.ShapeDtypeStruct((B, 1), jnp.float32),
    )(eug2, eig2, eum2, eim2, par_u, par_i, w1a_t, w1b_t, b1r, wg, wh, bp11)


def kernel(user, item, embed_user_GMF, embed_item_GMF, embed_user_MLP,
           embed_item_MLP, W1, b1, Wp, bp):
    user = user.astype(jnp.int32)
    item = item.astype(jnp.int32)
    pu = user >> 1
    pi = item >> 1
    par_u = (user & 1).reshape(B, 1)
    par_i = (item & 1).reshape(B, 1)
    tug2 = embed_user_GMF.reshape(-1, 2 * F)
    tig2 = embed_item_GMF.reshape(-1, 2 * F)
    tum2 = embed_user_MLP.reshape(-1, 2 * F)
    tim2 = embed_item_MLP.reshape(-1, 2 * F)
    eug2, eig2, eum2, eim2 = _sc_gather(pu, pi, tug2, tig2, tum2, tim2)
    w1a_t = W1[:, :F].T
    w1b_t = W1[:, F:].T
    b1r = b1.reshape(1, F)
    wg = Wp[0, :F].reshape(1, F)
    wh = Wp[0, F:].reshape(1, F)
    bp11 = bp.reshape(1, 1)
    out = _tc_head(eug2, eig2, eum2, eim2, par_u, par_i, w1a_t, w1b_t, b1r,
                   wg, wh, bp11)
    return out.reshape(B)
